# Initial kernel scaffold; baseline (speedup 1.0000x reference)
import jax
import jax.numpy as jnp
from jax.experimental import pallas as pl


def kernel(x, edge_index, train_edge_id, params):
    raise NotImplementedError("write your pallas kernel here")


# speedup vs baseline: 26.3914x; 26.3914x over previous
import functools

import jax
import jax.numpy as jnp
from jax.experimental import pallas as pl


N_NODES = 10000
N_EDGES = 160000


def _zero():
    return jnp.int32(0)


def _encoder_body(x_ref, w_ref, b_ref, o_ref):
    acc = jnp.dot(x_ref[...], w_ref[...], preferred_element_type=jnp.float32)
    o_ref[...] = jnp.maximum(acc + b_ref[...], 0.0)


def _encoder(x2, wc, bp):
    n = x2.shape[0]
    bn = 1000
    grid = (n // bn,)
    return pl.pallas_call(
        _encoder_body,
        grid=grid,
        in_specs=[
            pl.BlockSpec((bn, x2.shape[1]), lambda i: (i, _zero())),
            pl.BlockSpec((x2.shape[1], wc.shape[1]), lambda i: (_zero(), _zero())),
            pl.BlockSpec((1, wc.shape[1]), lambda i: (_zero(), _zero())),
        ],
        out_specs=pl.BlockSpec((bn, wc.shape[1]), lambda i: (i, _zero())),
        out_shape=jax.ShapeDtypeStruct((n, wc.shape[1]), jnp.float32),
    )(x2, wc, bp)


def _bn_apply(h, g, b):
    m = h.mean(axis=0)
    v = h.var(axis=0)
    return (h - m) / jnp.sqrt(v + jnp.float32(1e-5)) * g + b


def _bn_plain(h):
    m = h.mean(axis=0)
    v = h.var(axis=0)
    return (h - m) / jnp.sqrt(v + jnp.float32(1e-5))


def _gcn_branch(h, src, dst, n, W, b, dinv):
    hw = h @ W
    hp = hw * dinv[:, None]
    agg = jax.ops.segment_sum(hp[src], dst, num_segments=n)
    agg = dinv[:, None] * agg + dinv[:, None] * dinv[:, None] * hw
    return agg + b


def _gat_branch(h, src, dst, n, W, a_s, a_d, b, heads, oc):
    hh = (h @ W).reshape(n, heads, oc)
    al_s = (hh * a_s).sum(-1)
    al_d = (hh * a_d).sum(-1)
    m_node = jax.nn.leaky_relu(jnp.max(al_s, axis=0)[None, :] + al_d, 0.2)
    e = jax.nn.leaky_relu(al_s[src] + al_d[dst], 0.2)
    ee = jnp.exp(e - m_node[dst])
    e_self = jax.nn.leaky_relu(al_s + al_d, 0.2)
    ee_self = jnp.exp(e_self - m_node)
    denom = jax.ops.segment_sum(ee, dst, num_segments=n) + ee_self
    num = jax.ops.segment_sum(
        hh.reshape(n, heads * oc)[src]
        * jnp.repeat(ee, oc, axis=1).reshape(-1, heads * oc),
        dst,
        num_segments=n,
    )
    num = num + hh.reshape(n, heads * oc) * jnp.repeat(ee_self, oc, axis=1).reshape(
        n, heads * oc
    )
    attn_den = (denom + jnp.float32(1e-16)).reshape(n, heads, 1)
    out = (num.reshape(n, heads, oc) / attn_den).reshape(n, heads * oc)
    return out + b


def kernel(x, edge_index, train_edge_id, params):
    with jax.default_matmul_precision('highest'):
        return _impl(x, edge_index, train_edge_id, params)


def _impl(x, edge_index, train_edge_id, params):
    p = jax.tree.map(
        lambda a: a.astype(jnp.float32) if a.dtype == jnp.float64 else a, params)
    x = x.astype(jnp.float32)
    n = x.shape[0]
    src = edge_index[0]
    dst = edge_index[1]

    x2 = x.reshape(n, -1)
    wc = jnp.tile(p['Wp'], (x.shape[1], 1)) * jnp.float32(1.0 / x.shape[1])
    wc = wc.astype(jnp.float32)
    feat = _encoder(x2, wc, p['bp'][None, :])

    deg = jax.ops.segment_sum(jnp.ones(src.shape[0], jnp.float32), dst,
                              num_segments=n) + 1.0
    dinv = 1.0 / jnp.sqrt(deg)

    h = _gcn_branch(feat, src, dst, n, p['Wg1'], p['bg1'], dinv)
    h = _bn_apply(jax.nn.relu(h @ p['Wfc1g'] + p['bfc1g']), p['gam1g'], p['bet1g'])
    h = _gcn_branch(h, src, dst, n, p['Wg2'], p['bg2'], dinv)
    h = _bn_apply(jax.nn.relu(h @ p['Wfc2g'] + p['bfc2g']), p['gam2g'], p['bet2g'])
    g_out = jax.nn.relu(h @ p['Wling'] + p['bling'])

    fw = feat @ p['Wi1a']
    agg = fw + jax.ops.segment_sum(fw[src], dst, num_segments=n)
    h = jax.nn.relu(agg + p['bi1a'])
    h = jax.nn.relu(h @ p['Wi1b'] + p['bi1b'])
    h = _bn_plain(h) * p['gami1'] + p['beti1']
    hw = h @ p['Wi2a']
    agg = hw + jax.ops.segment_sum(hw[src], dst, num_segments=n)
    h = jax.nn.relu(agg + p['bi2a'])
    h = jax.nn.relu(h @ p['Wi2b'] + p['bi2b'])
    h = _bn_plain(h) * p['gami2'] + p['beti2']
    i_out = jax.nn.relu(h @ p['Wlini'] + p['blini'])

    h = _gat_branch(feat, src, dst, n, p['Wa1'], p['as1'], p['ad1'], p['ba1'], 8, 16)
    h = _bn_apply(jax.nn.relu(h @ p['Wfc1a'] + p['bfc1a']), p['gam1a'], p['bet1a'])
    h = _gat_branch(h, src, dst, n, p['Wa2'], p['as2'], p['ad2'], p['ba2'], 1, 128)
    h = _bn_apply(jax.nn.relu(h @ p['Wfc2a'] + p['bfc2a']), p['gam2a'], p['bet2a'])
    a_out = jax.nn.relu(h @ p['Wlina'] + p['blina'])

    s = p['wfuse'][0] * g_out + p['wfuse'][1] * i_out + p['wfuse'][2] * a_out
    s = _bn_plain(s)
    h = jax.nn.relu(s @ p['Wl1'] + p['bl1'])
    h = h @ p['Wl2'] + p['bl2']
    node_id = edge_index[:, train_edge_id]
    x1 = h[node_id[0]]
    x2o = h[node_id[1]]
    xm = x1 * x2o
    out = xm @ p['Wfc2'] + p['bfc2']
    return (out.astype(jnp.float64), x1.astype(jnp.float64),
            x2o.astype(jnp.float64), node_id[0], node_id[1])


# speedup vs baseline: 27.3772x; 1.0374x over previous
import functools

import jax
import jax.numpy as jnp
from jax import lax
from jax.experimental import pallas as pl
from jax.experimental.pallas import tpu as pltpu
from jax.experimental.pallas import tpu_sc as plsc


N_NODES = 10000
N_EDGES = 160000

_NC = 2
_NS = 16
_NW = _NC * _NS
_K = 128
_EPT = 5120
_EPAD = _EPT * _NW
_NCHUNK = _EPT // _K
_NPAD = 10112


def _zero():
    return jnp.int32(0)


def _seg_body(tab, srcp, dstp, zz, out, sidx, didx, rows, acc, sem):
    c = lax.axis_index("c")
    s = lax.axis_index("s")
    wid = c * _NS + s
    rows_per = _NPAD // _NS
    rbase = s * rows_per
    pltpu.sync_copy(zz.at[pl.ds(rbase, rows_per)], acc.at[pl.ds(rbase, rows_per)])
    plsc.subcore_barrier()

    def chunk(i, carry):
        eb = (wid * _EPT + i * _K).astype(jnp.int32)
        pltpu.sync_copy(srcp.at[pl.ds(eb, _K)], sidx)
        pltpu.sync_copy(dstp.at[pl.ds(eb, _K)], didx)
        pltpu.async_copy(tab.at[sidx], rows, sem).wait()
        pltpu.sync_copy(rows, acc.at[didx], add=True)
        return carry

    lax.fori_loop(jnp.int32(0), jnp.int32(_NCHUNK), chunk, jnp.int32(0))
    plsc.subcore_barrier()
    pltpu.sync_copy(acc.at[pl.ds(rbase, rows_per)],
                    out.at[c, pl.ds(rbase, rows_per)])


def _segsum_sc(tab, src_p, dst_p, zz):
    mesh = plsc.VectorSubcoreMesh(core_axis_name="c", subcore_axis_name="s")
    fn = pl.kernel(
        _seg_body,
        out_type=jax.ShapeDtypeStruct((_NC, _NPAD, 128), jnp.float32),
        mesh=mesh,
        scratch_types=[
            pltpu.VMEM((_K,), jnp.int32),
            pltpu.VMEM((_K,), jnp.int32),
            pltpu.VMEM((_K, 128), jnp.float32),
            pltpu.VMEM_SHARED((_NPAD, 128), jnp.float32),
            pltpu.SemaphoreType.DMA,
        ],
    )
    return fn(tab, src_p, dst_p, zz)


def _pad_edges(src, dst):
    npad = _EPAD - src.shape[0]
    src_p = jnp.concatenate([src, jnp.zeros((npad,), jnp.int32)])
    dst_p = jnp.concatenate([dst, jnp.full((npad,), N_NODES, jnp.int32)])
    return src_p, dst_p


def _seg_deg_body(tab, srcp, dstp, zz, zz16, out, dout,
                  sidx, didx, rows, ones, acc, acc16, sem):
    c = lax.axis_index("c")
    s = lax.axis_index("s")
    wid = c * _NS + s
    rows_per = _NPAD // _NS
    rbase = s * rows_per
    pltpu.sync_copy(zz.at[pl.ds(rbase, rows_per)], acc.at[pl.ds(rbase, rows_per)])
    pltpu.sync_copy(zz16.at[pl.ds(rbase, rows_per)],
                    acc16.at[pl.ds(rbase, rows_per)])

    def fill(i, carry):
        ones[i] = jnp.ones((16,), jnp.float32)
        return carry

    lax.fori_loop(jnp.int32(0), jnp.int32(_K), fill, jnp.int32(0))
    plsc.subcore_barrier()

    def chunk(i, carry):
        eb = (wid * _EPT + i * _K).astype(jnp.int32)
        pltpu.sync_copy(srcp.at[pl.ds(eb, _K)], sidx)
        pltpu.sync_copy(dstp.at[pl.ds(eb, _K)], didx)
        pltpu.async_copy(tab.at[sidx], rows, sem).wait()
        pltpu.sync_copy(rows, acc.at[didx], add=True)
        pltpu.sync_copy(ones, acc16.at[didx], add=True)
        return carry

    lax.fori_loop(jnp.int32(0), jnp.int32(_NCHUNK), chunk, jnp.int32(0))
    plsc.subcore_barrier()
    pltpu.sync_copy(acc.at[pl.ds(rbase, rows_per)],
                    out.at[c, pl.ds(rbase, rows_per)])
    pltpu.sync_copy(acc16.at[pl.ds(rbase, rows_per)],
                    dout.at[c, pl.ds(rbase, rows_per)])


def _segsum_deg_sc(tab, src_p, dst_p, zz, zz16):
    mesh = plsc.VectorSubcoreMesh(core_axis_name="c", subcore_axis_name="s")
    fn = pl.kernel(
        _seg_deg_body,
        out_type=(jax.ShapeDtypeStruct((_NC, _NPAD, 128), jnp.float32),
                  jax.ShapeDtypeStruct((_NC, _NPAD, 16), jnp.float32)),
        mesh=mesh,
        scratch_types=[
            pltpu.VMEM((_K,), jnp.int32),
            pltpu.VMEM((_K,), jnp.int32),
            pltpu.VMEM((_K, 128), jnp.float32),
            pltpu.VMEM((_K, 16), jnp.float32),
            pltpu.VMEM_SHARED((_NPAD, 128), jnp.float32),
            pltpu.VMEM_SHARED((_NPAD, 16), jnp.float32),
            pltpu.SemaphoreType.DMA,
        ],
    )
    return fn(tab, src_p, dst_p, zz, zz16)


_PERM8 = tuple(list(range(8)) + list(range(8)))


def _gat_body(stab, atab, dtab, srcp, dstp, zz, zz16, out, dout,
              sidx, didx, strows, arows, drows, scbuf, eebuf, acc, acc16, sem):
    c = lax.axis_index("c")
    s = lax.axis_index("s")
    wid = c * _NS + s
    rows_per = _NPAD // _NS
    rbase = s * rows_per
    pltpu.sync_copy(zz.at[pl.ds(rbase, rows_per)], acc.at[pl.ds(rbase, rows_per)])
    pltpu.sync_copy(zz16.at[pl.ds(rbase, rows_per)],
                    acc16.at[pl.ds(rbase, rows_per)])
    plsc.subcore_barrier()
    perm = jnp.array(_PERM8, jnp.int32)

    def chunk(i, carry):
        eb = (wid * _EPT + i * _K).astype(jnp.int32)
        pltpu.sync_copy(srcp.at[pl.ds(eb, _K)], sidx)
        pltpu.sync_copy(dstp.at[pl.ds(eb, _K)], didx)
        pltpu.async_copy(stab.at[sidx], strows, sem).wait()
        pltpu.async_copy(atab.at[sidx], arows, sem).wait()
        pltpu.async_copy(dtab.at[didx], drows, sem).wait()

        def edge(k, carry2):
            av = arows[k, :]
            bv = drows[k, pl.ds(0, 16)]
            mv = drows[k, pl.ds(16, 16)]
            z = av + bv
            l = jnp.where(z > 0, z, z * jnp.float32(0.2))
            ee = jnp.exp(l - mv)
            eebuf[k, :] = ee
            ev = plsc.load_gather(eebuf, [jnp.full((16,), k, jnp.int32), perm])
            for q in range(8):
                scbuf[k, pl.ds(16 * q, 16)] = strows[k, pl.ds(16 * q, 16)] * ev
            eebuf[k, :] = ev
            return carry2

        lax.fori_loop(jnp.int32(0), jnp.int32(_K), edge, jnp.int32(0))
        pltpu.sync_copy(scbuf, acc.at[didx], add=True)
        pltpu.sync_copy(eebuf, acc16.at[didx], add=True)
        return carry

    lax.fori_loop(jnp.int32(0), jnp.int32(_NCHUNK), chunk, jnp.int32(0))
    plsc.subcore_barrier()
    pltpu.sync_copy(acc.at[pl.ds(rbase, rows_per)],
                    out.at[c, pl.ds(rbase, rows_per)])
    pltpu.sync_copy(acc16.at[pl.ds(rbase, rows_per)],
                    dout.at[c, pl.ds(rbase, rows_per)])


def _gat_sc(stab, atab, dtab, src_p, dst_p, zz, zz16):
    mesh = plsc.VectorSubcoreMesh(core_axis_name="c", subcore_axis_name="s")
    fn = pl.kernel(
        _gat_body,
        out_type=(jax.ShapeDtypeStruct((_NC, _NPAD, 128), jnp.float32),
                  jax.ShapeDtypeStruct((_NC, _NPAD, 16), jnp.float32)),
        mesh=mesh,
        scratch_types=[
            pltpu.VMEM((_K,), jnp.int32),
            pltpu.VMEM((_K,), jnp.int32),
            pltpu.VMEM((_K, 128), jnp.float32),
            pltpu.VMEM((_K, 16), jnp.float32),
            pltpu.VMEM((_K, 32), jnp.float32),
            pltpu.VMEM((_K, 128), jnp.float32),
            pltpu.VMEM((_K, 16), jnp.float32),
            pltpu.VMEM_SHARED((_NPAD, 128), jnp.float32),
            pltpu.VMEM_SHARED((_NPAD, 16), jnp.float32),
            pltpu.SemaphoreType.DMA,
        ],
    )
    return fn(stab, atab, dtab, src_p, dst_p, zz, zz16)


_GPAD = 40960
_GSEG = 20480


def _gather_body(tab, ids, out, iidx, rows, sem):
    wid = lax.axis_index("c") * _NS + lax.axis_index("s")
    per = _GPAD // _NW

    def chunk(i, carry):
        eb = (wid * per + i * _K).astype(jnp.int32)
        pltpu.sync_copy(ids.at[pl.ds(eb, _K)], iidx)
        pltpu.async_copy(tab.at[iidx], rows, sem).wait()
        pltpu.sync_copy(rows, out.at[pl.ds(eb, _K)])
        return carry

    lax.fori_loop(jnp.int32(0), jnp.int32(per // _K), chunk, jnp.int32(0))


def _gather_sc(tab, ids):
    mesh = plsc.VectorSubcoreMesh(core_axis_name="c", subcore_axis_name="s")
    fn = pl.kernel(
        _gather_body,
        out_type=jax.ShapeDtypeStruct((_GPAD, 512), jnp.float32),
        mesh=mesh,
        scratch_types=[
            pltpu.VMEM((_K,), jnp.int32),
            pltpu.VMEM((_K, 512), jnp.float32),
            pltpu.SemaphoreType.DMA,
        ],
    )
    return fn(tab, ids)


def _encoder_body(x_ref, w_ref, b_ref, o_ref):
    acc = jnp.dot(x_ref[...], w_ref[...], preferred_element_type=jnp.float32)
    o_ref[...] = jnp.maximum(acc + b_ref[...], 0.0)


def _encoder(x2, wc, bp):
    n = x2.shape[0]
    bn = 1000
    grid = (n // bn,)
    return pl.pallas_call(
        _encoder_body,
        grid=grid,
        in_specs=[
            pl.BlockSpec((bn, x2.shape[1]), lambda i: (i, _zero())),
            pl.BlockSpec((x2.shape[1], wc.shape[1]), lambda i: (_zero(), _zero())),
            pl.BlockSpec((1, wc.shape[1]), lambda i: (_zero(), _zero())),
        ],
        out_specs=pl.BlockSpec((bn, wc.shape[1]), lambda i: (i, _zero())),
        out_shape=jax.ShapeDtypeStruct((n, wc.shape[1]), jnp.float32),
    )(x2, wc, bp)


def _bn_apply(h, g, b):
    m = h.mean(axis=0)
    v = h.var(axis=0)
    return (h - m) / jnp.sqrt(v + jnp.float32(1e-5)) * g + b


def _bn_plain(h):
    m = h.mean(axis=0)
    v = h.var(axis=0)
    return (h - m) / jnp.sqrt(v + jnp.float32(1e-5))


def _gcn_branch(h, W, b, dinv, segsum128):
    hw = h @ W
    hp = hw * dinv[:, None]
    agg = segsum128(hp)
    agg = dinv[:, None] * agg + dinv[:, None] * dinv[:, None] * hw
    return agg + b


def _gat_branch(h, src, dst, n, W, a_s, a_d, b, heads, oc):
    hh = (h @ W).reshape(n, heads, oc)
    al_s = (hh * a_s).sum(-1)
    al_d = (hh * a_d).sum(-1)
    m_node = jax.nn.leaky_relu(jnp.max(al_s, axis=0)[None, :] + al_d, 0.2)
    e = jax.nn.leaky_relu(al_s[src] + al_d[dst], 0.2)
    ee = jnp.exp(e - m_node[dst])
    e_self = jax.nn.leaky_relu(al_s + al_d, 0.2)
    ee_self = jnp.exp(e_self - m_node)
    denom = jax.ops.segment_sum(ee, dst, num_segments=n) + ee_self
    num = jax.ops.segment_sum(
        hh.reshape(n, heads * oc)[src]
        * jnp.repeat(ee, oc, axis=1).reshape(-1, heads * oc),
        dst,
        num_segments=n,
    )
    num = num + hh.reshape(n, heads * oc) * jnp.repeat(ee_self, oc, axis=1).reshape(
        n, heads * oc
    )
    attn_den = (denom + jnp.float32(1e-16)).reshape(n, heads, 1)
    out = (num.reshape(n, heads, oc) / attn_den).reshape(n, heads * oc)
    return out + b


def kernel(x, edge_index, train_edge_id, params):
    with jax.default_matmul_precision('highest'):
        return _impl(x, edge_index, train_edge_id, params)


def _impl(x, edge_index, train_edge_id, params):
    p = jax.tree.map(
        lambda a: a.astype(jnp.float32) if a.dtype == jnp.float64 else a, params)
    x = x.astype(jnp.float32)
    n = x.shape[0]
    src = edge_index[0]
    dst = edge_index[1]

    x2 = x.reshape(n, -1)
    wc = jnp.tile(p['Wp'], (x.shape[1], 1)) * jnp.float32(1.0 / x.shape[1])
    wc = wc.astype(jnp.float32)
    feat = _encoder(x2, wc, p['bp'][None, :])

    src32 = src.astype(jnp.int32)
    dst32 = dst.astype(jnp.int32)
    src_p, dst_p = _pad_edges(src32, dst32)
    zz = jnp.zeros((_NPAD, 128), jnp.float32)

    def segsum128(tab):
        ps = _segsum_sc(tab, src_p, dst_p, zz)
        return (ps[0] + ps[1])[:N_NODES]

    deg = jax.ops.segment_sum(jnp.ones(src.shape[0], jnp.float32), dst,
                              num_segments=n) + 1.0
    dinv = 1.0 / jnp.sqrt(deg)

    h = _gcn_branch(feat, p['Wg1'], p['bg1'], dinv, segsum128)
    h = _bn_apply(jax.nn.relu(h @ p['Wfc1g'] + p['bfc1g']), p['gam1g'], p['bet1g'])
    h = _gcn_branch(h, p['Wg2'], p['bg2'], dinv, segsum128)
    h = _bn_apply(jax.nn.relu(h @ p['Wfc2g'] + p['bfc2g']), p['gam2g'], p['bet2g'])
    g_out = jax.nn.relu(h @ p['Wling'] + p['bling'])

    fw = feat @ p['Wi1a']
    agg = fw + segsum128(fw)
    h = jax.nn.relu(agg + p['bi1a'])
    h = jax.nn.relu(h @ p['Wi1b'] + p['bi1b'])
    h = _bn_plain(h) * p['gami1'] + p['beti1']
    hw = h @ p['Wi2a']
    agg = hw + segsum128(hw)
    h = jax.nn.relu(agg + p['bi2a'])
    h = jax.nn.relu(h @ p['Wi2b'] + p['bi2b'])
    h = _bn_plain(h) * p['gami2'] + p['beti2']
    i_out = jax.nn.relu(h @ p['Wlini'] + p['blini'])

    h = _gat_branch(feat, src, dst, n, p['Wa1'], p['as1'], p['ad1'], p['ba1'], 8, 16)
    h = _bn_apply(jax.nn.relu(h @ p['Wfc1a'] + p['bfc1a']), p['gam1a'], p['bet1a'])
    h = _gat_branch(h, src, dst, n, p['Wa2'], p['as2'], p['ad2'], p['ba2'], 1, 128)
    h = _bn_apply(jax.nn.relu(h @ p['Wfc2a'] + p['bfc2a']), p['gam2a'], p['bet2a'])
    a_out = jax.nn.relu(h @ p['Wlina'] + p['blina'])

    s = p['wfuse'][0] * g_out + p['wfuse'][1] * i_out + p['wfuse'][2] * a_out
    s = _bn_plain(s)
    h = jax.nn.relu(s @ p['Wl1'] + p['bl1'])
    h = h @ p['Wl2'] + p['bl2']
    node_id = edge_index[:, train_edge_id]
    x1 = h[node_id[0]]
    x2o = h[node_id[1]]
    xm = x1 * x2o
    out = xm @ p['Wfc2'] + p['bfc2']
    return (out.astype(jnp.float64), x1.astype(jnp.float64),
            x2o.astype(jnp.float64), node_id[0], node_id[1])
